# baseline (speedup 1.0000x reference)
import functools

import jax
import jax.numpy as jnp
import numpy as np
from jax import lax
from jax.experimental import pallas as pl
from jax.experimental.pallas import tpu as pltpu
from jax.experimental.pallas import tpu_sc as plsc

H = 1024
NH = 4
HD = H // NH
E = 8
I_QG = 5632
I_CR = 2816
I_GN = 2816
THETA = 10000.0
S = 2048
BLK = 256
NB_A, NB_B, NB_C = 16, 16, 20
NB = NB_A + NB_B + NB_C
P = NB * BLK
KC = 11
KA, KB = 512, 256

_NC, _NS = 2, 16
_NW = _NC * _NS


def _rope_perm():
    p = np.arange(H).reshape(NH, HD)
    out = np.concatenate([p[:, 0::2], p[:, 1::2]], axis=1).reshape(-1)
    return out


def _rope_perm_k():
    p = np.arange(HD)
    return np.concatenate([p[0::2], p[1::2]])


def _s1_body(x_ref, qw_ref, kw_ref, vw_ref, g_ref, b_ref, cos_ref, sin_ref,
             q_ref, k_ref, v_ref):
    x = x_ref[...]
    mu = jnp.mean(x, axis=1, keepdims=True)
    var = jnp.mean((x - mu) ** 2, axis=1, keepdims=True)
    hn = (x - mu) * lax.rsqrt(var + 1e-5) * g_ref[...] + b_ref[...]
    q = jnp.dot(hn, qw_ref[...], preferred_element_type=jnp.float32)
    k = jnp.dot(hn, kw_ref[...], preferred_element_type=jnp.float32)
    v = jnp.dot(hn, vw_ref[...], preferred_element_type=jnp.float32)
    c = cos_ref[...]
    s = sin_ref[...]
    qo = []
    for h in range(NH):
        q1 = q[:, h * HD:h * HD + HD // 2]
        q2 = q[:, h * HD + HD // 2:(h + 1) * HD]
        qo.append(q1 * c - q2 * s)
        qo.append(q2 * c + q1 * s)
    q_ref[...] = jnp.concatenate(qo, axis=1)
    k1 = k[:, :HD // 2]
    k2 = k[:, HD // 2:]
    k_ref[...] = jnp.concatenate([k1 * c - k2 * s, k2 * c + k1 * s], axis=1)
    v_ref[...] = v


def _stage1(x, qwp, kwp, vw, g, b, cos, sin):
    n = S // BLK
    return pl.pallas_call(
        _s1_body,
        grid=(n,),
        in_specs=[
            pl.BlockSpec((BLK, H), lambda i: (i, 0)),
            pl.BlockSpec((H, H), lambda i: (0, 0)),
            pl.BlockSpec((H, HD), lambda i: (0, 0)),
            pl.BlockSpec((H, HD), lambda i: (0, 0)),
            pl.BlockSpec((1, H), lambda i: (0, 0)),
            pl.BlockSpec((1, H), lambda i: (0, 0)),
            pl.BlockSpec((BLK, HD // 2), lambda i: (i, 0)),
            pl.BlockSpec((BLK, HD // 2), lambda i: (i, 0)),
        ],
        out_specs=[
            pl.BlockSpec((BLK, H), lambda i: (i, 0)),
            pl.BlockSpec((BLK, HD), lambda i: (i, 0)),
            pl.BlockSpec((BLK, HD), lambda i: (i, 0)),
        ],
        out_shape=[
            jax.ShapeDtypeStruct((S, H), jnp.float32),
            jax.ShapeDtypeStruct((S, HD), jnp.float32),
            jax.ShapeDtypeStruct((S, HD), jnp.float32),
        ],
    )(x, qwp, kwp, vw, g, b, cos, sin)


def _attn_body(q_ref, k_ref, v_ref, o_ref):
    q = q_ref[...]
    k = k_ref[...]
    s = lax.dot_general(q, k, (((1,), (1,)), ((), ())),
                        preferred_element_type=jnp.float32) * (HD ** -0.5)
    m = jnp.max(s, axis=1, keepdims=True)
    p = jnp.exp(s - m)
    l = jnp.sum(p, axis=1, keepdims=True)
    o = jnp.dot(p, v_ref[...], preferred_element_type=jnp.float32)
    o_ref[...] = o / l


def _attn(q, k, v):
    n = S // BLK
    return pl.pallas_call(
        _attn_body,
        grid=(NH, n),
        in_specs=[
            pl.BlockSpec((BLK, HD), lambda h, i: (i, h)),
            pl.BlockSpec((S, HD), lambda h, i: (0, 0)),
            pl.BlockSpec((S, HD), lambda h, i: (0, 0)),
        ],
        out_specs=pl.BlockSpec((BLK, HD), lambda h, i: (i, h)),
        out_shape=jax.ShapeDtypeStruct((S, H), jnp.float32),
    )(q, k, v)


def _s3_body(a_ref, ow_ref, r_ref, g_ref, b_ref, rw_ref,
             h2_ref, hn_ref, w_ref):
    ao = jnp.dot(a_ref[...], ow_ref[...], preferred_element_type=jnp.float32)
    h2 = r_ref[...] + ao
    h2_ref[...] = h2
    mu = jnp.mean(h2, axis=1, keepdims=True)
    var = jnp.mean((h2 - mu) ** 2, axis=1, keepdims=True)
    hn = (h2 - mu) * lax.rsqrt(var + 1e-5) * g_ref[...] + b_ref[...]
    hn_ref[...] = hn
    logits = jnp.dot(hn, rw_ref[...], preferred_element_type=jnp.float32)
    lane = lax.broadcasted_iota(jnp.int32, logits.shape, 1)
    valid = lane < E
    logits = jnp.where(valid, logits, -1e30)
    m = jnp.max(logits, axis=1, keepdims=True)
    ex = jnp.where(valid, jnp.exp(logits - m), 0.0)
    probs = ex / jnp.sum(ex, axis=1, keepdims=True)
    m1 = jnp.max(probs, axis=1, keepdims=True)
    i1 = jnp.min(jnp.where(probs == m1, lane, 128), axis=1, keepdims=True)
    p2 = jnp.where(lane == i1, -1.0, probs)
    m2 = jnp.max(p2, axis=1, keepdims=True)
    i2 = jnp.min(jnp.where(p2 == m2, lane, 128), axis=1, keepdims=True)
    tot = m1 + m2
    w1 = m1 / tot
    w2 = m2 / tot
    w_ref[...] = jnp.where(lane == i1, w1, 0.0) + jnp.where(lane == i2, w2, 0.0)


def _stage3(attn_out, ow, resid, g, b, rwp):
    n = S // BLK
    return pl.pallas_call(
        _s3_body,
        grid=(n,),
        in_specs=[
            pl.BlockSpec((BLK, H), lambda i: (i, 0)),
            pl.BlockSpec((H, H), lambda i: (0, 0)),
            pl.BlockSpec((BLK, H), lambda i: (i, 0)),
            pl.BlockSpec((1, H), lambda i: (0, 0)),
            pl.BlockSpec((1, H), lambda i: (0, 0)),
            pl.BlockSpec((H, 128), lambda i: (0, 0)),
        ],
        out_specs=[
            pl.BlockSpec((BLK, H), lambda i: (i, 0)),
            pl.BlockSpec((BLK, H), lambda i: (i, 0)),
            pl.BlockSpec((BLK, 128), lambda i: (i, 0)),
        ],
        out_shape=[
            jax.ShapeDtypeStruct((S, H), jnp.float32),
            jax.ShapeDtypeStruct((S, H), jnp.float32),
            jax.ShapeDtypeStruct((S, 128), jnp.float32),
        ],
    )(attn_out, ow, resid, g, b, rwp)


def _sc_gather(src, table):
    rows_pw = P // _NW
    nch = rows_pw // 32
    mesh = plsc.VectorSubcoreMesh(core_axis_name="c", subcore_axis_name="s")

    @functools.partial(
        pl.kernel, mesh=mesh,
        out_type=jax.ShapeDtypeStruct((P, H), jnp.float32),
        scratch_types=[
            pltpu.VMEM((32,), jnp.int32),
            pltpu.VMEM((32, H), jnp.float32),
            pltpu.SemaphoreType.DMA,
        ],
    )
    def k(src_hbm, table_hbm, out_hbm, idx_v, rows_v, sem):
        wid = lax.axis_index("s") * _NC + lax.axis_index("c")
        base = wid * rows_pw
        for c in range(nch):
            off = base + c * 32
            pltpu.sync_copy(src_hbm.at[pl.ds(off, 32)], idx_v)
            pltpu.async_copy(table_hbm.at[idx_v], rows_v, sem).wait()
            pltpu.sync_copy(rows_v, out_hbm.at[pl.ds(off, 32)])

    return k(src, table)


def _sc_combine(y, resid2, pos2):
    toks_pw = S // _NW
    mesh = plsc.VectorSubcoreMesh(core_axis_name="c", subcore_axis_name="s")

    @functools.partial(
        pl.kernel, mesh=mesh,
        out_type=jax.ShapeDtypeStruct((S, H), jnp.float32),
        scratch_types=[
            pltpu.VMEM((32,), jnp.int32),
            pltpu.VMEM((32, H), jnp.float32),
            pltpu.VMEM((16, H), jnp.float32),
            pltpu.SemaphoreType.DMA,
        ],
    )
    def k(y_hbm, r_hbm, pos_hbm, out_hbm, idx_v, rows_v, acc_v, sem):
        wid = lax.axis_index("s") * _NC + lax.axis_index("c")
        for c in range(toks_pw // 16):
            t0 = wid * toks_pw + c * 16
            pltpu.sync_copy(pos_hbm.at[pl.ds(t0 * 2, 32)], idx_v)
            pltpu.async_copy(y_hbm.at[idx_v], rows_v, sem).wait()
            pltpu.sync_copy(r_hbm.at[pl.ds(t0, 16)], acc_v)

            def tok_body(i, _):
                def d_body(d, _):
                    sl = pl.ds(d * 16, 16)
                    acc_v[i, sl] = (acc_v[i, sl] + rows_v[2 * i, sl]
                                    + rows_v[2 * i + 1, sl])
                    return 0
                lax.fori_loop(0, H // 16, d_body, 0)
                return 0

            lax.fori_loop(0, 16, tok_body, 0)
            pltpu.sync_copy(acc_v, out_hbm.at[pl.ds(t0, 16)])

    return k(y, resid2, pos2)


def _moe_body(act_ref, beA_ref, jA_ref, beB_ref, jB_ref, beC_ref, jC_ref,
              x_ref, w_ref,
              ga_ref, ua_ref, da_ref,
              gb_ref, ub_ref, db_ref, cr_ref,
              gc_ref, uc_ref, dc_ref,
              y_ref, acc_ref):
    i = pl.program_id(0)
    j = pl.program_id(1)
    act = act_ref[i]

    @pl.when(act == 1)
    def _():
        x = x_ref[...]

        def fam(g_r, u_r, d_r, scale, extra):
            g = jnp.dot(x, g_r[0], preferred_element_type=jnp.float32)
            u = jnp.dot(x, u_r[0], preferred_element_type=jnp.float32)
            t = g * lax.logistic(g) * u * scale
            part = jnp.dot(t, d_r[0], preferred_element_type=jnp.float32)
            if extra is not None:
                part = part + extra

            @pl.when(j == 0)
            def _():
                acc_ref[...] = part

            @pl.when(j > 0)
            def _():
                acc_ref[...] = acc_ref[...] + part

        @pl.when(i < NB_A)
        def _():
            fam(ga_ref, ua_ref, da_ref, 1.1, None)

        @pl.when((i >= NB_A) & (i < NB_A + NB_B))
        def _():
            cre = jnp.where(
                j == 0,
                jnp.tanh(jnp.dot(x, cr_ref[0],
                                 preferred_element_type=jnp.float32)) * 0.2,
                0.0)
            fam(gb_ref, ub_ref, db_ref, 1.0, cre)

        @pl.when(i >= NB_A + NB_B)
        def _():
            fam(gc_ref, uc_ref, dc_ref, 1.0, None)

        @pl.when(j == KC - 1)
        def _():
            y_ref[...] = acc_ref[...] * w_ref[:, :1]


def _moe(xg, w2d, meta, qg_gate, qg_up, qg_down, cr_gate, cr_up, cr_down,
         cr_creative, gn_gate, gn_up, gn_down):
    act, beA, jA, beB, jB, beC, jC = meta
    grid_spec = pltpu.PrefetchScalarGridSpec(
        num_scalar_prefetch=7,
        grid=(NB, KC),
        in_specs=[
            pl.BlockSpec((BLK, H), lambda i, j, *p: (i, 0)),
            pl.BlockSpec((BLK, 128), lambda i, j, *p: (i, 0)),
            pl.BlockSpec((1, H, KA), lambda i, j, a, bA, jA, bB, jB, bC, jC: (bA[i], 0, j * jA[i])),
            pl.BlockSpec((1, H, KA), lambda i, j, a, bA, jA, bB, jB, bC, jC: (bA[i], 0, j * jA[i])),
            pl.BlockSpec((1, KA, H), lambda i, j, a, bA, jA, bB, jB, bC, jC: (bA[i], j * jA[i], 0)),
            pl.BlockSpec((1, H, KB), lambda i, j, a, bA, jA, bB, jB, bC, jC: (bB[i], 0, j * jB[i])),
            pl.BlockSpec((1, H, KB), lambda i, j, a, bA, jA, bB, jB, bC, jC: (bB[i], 0, j * jB[i])),
            pl.BlockSpec((1, KB, H), lambda i, j, a, bA, jA, bB, jB, bC, jC: (bB[i], j * jB[i], 0)),
            pl.BlockSpec((1, H, H), lambda i, j, a, bA, jA, bB, jB, bC, jC: (bB[i], 0, 0)),
            pl.BlockSpec((1, H, KB), lambda i, j, a, bA, jA, bB, jB, bC, jC: (bC[i], 0, j * jC[i])),
            pl.BlockSpec((1, H, KB), lambda i, j, a, bA, jA, bB, jB, bC, jC: (bC[i], 0, j * jC[i])),
            pl.BlockSpec((1, KB, H), lambda i, j, a, bA, jA, bB, jB, bC, jC: (bC[i], j * jC[i], 0)),
        ],
        out_specs=pl.BlockSpec((BLK, H), lambda i, j, *p: (i, 0)),
        scratch_shapes=[pltpu.VMEM((BLK, H), jnp.float32)],
    )
    return pl.pallas_call(
        _moe_body,
        grid_spec=grid_spec,
        out_shape=jax.ShapeDtypeStruct((P, H), jnp.float32),
    )(act, beA, jA, beB, jB, beC, jC,
      xg, w2d, qg_gate, qg_up, qg_down, cr_gate, cr_up, cr_down, cr_creative,
      gn_gate, gn_up, gn_down)


def _routing(w_full):
    wf = w_full[:, :E]
    sel = wf > 0.0
    seli = sel.astype(jnp.int32)
    counts = jnp.sum(seli, axis=0)
    blocks = (counts + BLK - 1) // BLK
    base = jnp.array([0, 0, NB_A, NB_A, NB_A + NB_B, 0, 0, 0], jnp.int32)
    off_blk = jnp.stack([
        base[0], blocks[0],
        base[2], base[2] + blocks[2],
        base[4], base[4] + blocks[4],
        base[4] + blocks[4] + blocks[5],
        base[4] + blocks[4] + blocks[5] + blocks[6],
    ])
    padded_off = off_blk * BLK
    rank = jnp.cumsum(seli, axis=0) - seli
    pos_te = padded_off[None, :] + rank
    tok = jnp.broadcast_to(jnp.arange(S, dtype=jnp.int32)[:, None], (S, E))
    ps = jnp.where(sel, pos_te, P).astype(jnp.int32).ravel()
    src = jnp.zeros((P + 1,), jnp.int32).at[ps].set(tok.ravel())[:P]
    w_pad = jnp.zeros((P + 1,), jnp.float32).at[ps].set(wf.ravel())[:P]
    w2d = jnp.broadcast_to(w_pad[:, None], (P, 128))
    e0 = jnp.argmax(seli, axis=1)
    e1 = (E - 1) - jnp.argmax(seli[:, ::-1], axis=1)
    pos2 = jnp.stack([
        jnp.take_along_axis(pos_te, e0[:, None], axis=1)[:, 0],
        jnp.take_along_axis(pos_te, e1[:, None], axis=1)[:, 0],
    ], axis=1).astype(jnp.int32).ravel()

    bi = jnp.arange(NB, dtype=jnp.int32)
    nA = blocks[0] + blocks[1]
    actA = (bi < nA).astype(jnp.int32)
    beA = jnp.where((bi >= blocks[0]) & (bi < nA), 1, 0).astype(jnp.int32)
    bl = bi - NB_A
    nBf = blocks[2] + blocks[3]
    actB = ((bi >= NB_A) & (bl < nBf)).astype(jnp.int32)
    beB = jnp.where(actB * (bl >= blocks[2]) == 1, 1, 0).astype(jnp.int32)
    cl = bi - NB_A - NB_B
    c4, c5, c6, c7 = blocks[4], blocks[5], blocks[6], blocks[7]
    nC = c4 + c5 + c6 + c7
    actC = ((bi >= NB_A + NB_B) & (cl < nC)).astype(jnp.int32)
    beC = ((cl >= c4).astype(jnp.int32) + (cl >= c4 + c5).astype(jnp.int32)
           + (cl >= c4 + c5 + c6).astype(jnp.int32)) * actC
    act = actA + actB + actC
    jA = actA
    jB = actB
    jC = actC
    return src, w2d, pos2, (act, beA, jA, beB, jB, beC, jC)


def kernel(hidden_states, q_w, k_w, v_w, o_w, ln1_g, ln1_b, ln2_g, ln2_b,
           router_w, qg_gate, qg_up, qg_down, cr_gate, cr_up, cr_down,
           cr_creative, gn_gate, gn_up, gn_down):
    x = hidden_states.reshape(S, H)
    qwp = q_w[:, _rope_perm()]
    kwp = k_w[:, _rope_perm_k()]
    freqs = 1.0 / THETA ** (jnp.arange(0, HD, 2, dtype=jnp.float32) / HD)
    t = jnp.arange(S, dtype=jnp.float32)
    ang = jnp.outer(t, freqs)
    cos = jnp.cos(ang)
    sin = jnp.sin(ang)
    rwp = jnp.pad(router_w, ((0, 0), (0, 128 - E)))

    q, k, v = _stage1(x, qwp, kwp, v_w, ln1_g[None, :], ln1_b[None, :], cos, sin)
    attn_out = _attn(q, k, v)
    h2, hn2, w_full = _stage3(attn_out, o_w, x, ln2_g[None, :], ln2_b[None, :], rwp)
    src, w2d, pos2, meta = _routing(w_full)
    xg = _sc_gather(src, hn2)
    y = _moe(xg, w2d, meta, qg_gate, qg_up, qg_down, cr_gate, cr_up, cr_down,
             cr_creative, gn_gate, gn_up, gn_down)
    out = _sc_combine(y, h2, pos2)
    return out.reshape(1, S, H)


# speedup vs baseline: 1.3846x; 1.3846x over previous
import functools

import jax
import jax.numpy as jnp
import numpy as np
from jax import lax
from jax.experimental import pallas as pl
from jax.experimental.pallas import tpu as pltpu
from jax.experimental.pallas import tpu_sc as plsc

H = 1024
NH = 4
HD = H // NH
E = 8
I_QG = 5632
I_CR = 2816
I_GN = 2816
THETA = 10000.0
S = 2048
BLK = 256
NB_A, NB_B, NB_C = 16, 16, 20
NB = NB_A + NB_B + NB_C
P = NB * BLK
KC = 11
KA, KB = 512, 256

_NC, _NS = 2, 16
_NW = _NC * _NS


def _rope_perm():
    p = np.arange(H).reshape(NH, HD)
    out = np.concatenate([p[:, 0::2], p[:, 1::2]], axis=1).reshape(-1)
    return out


def _rope_perm_k():
    p = np.arange(HD)
    return np.concatenate([p[0::2], p[1::2]])


def _s1_body(x_ref, qw_ref, kw_ref, vw_ref, g_ref, b_ref, cos_ref, sin_ref,
             q_ref, k_ref, v_ref):
    x = x_ref[...]
    mu = jnp.mean(x, axis=1, keepdims=True)
    var = jnp.mean((x - mu) ** 2, axis=1, keepdims=True)
    hn = (x - mu) * lax.rsqrt(var + 1e-5) * g_ref[...] + b_ref[...]
    q = jnp.dot(hn, qw_ref[...], preferred_element_type=jnp.float32)
    k = jnp.dot(hn, kw_ref[...], preferred_element_type=jnp.float32)
    v = jnp.dot(hn, vw_ref[...], preferred_element_type=jnp.float32)
    c = cos_ref[...]
    s = sin_ref[...]
    qo = []
    for h in range(NH):
        q1 = q[:, h * HD:h * HD + HD // 2]
        q2 = q[:, h * HD + HD // 2:(h + 1) * HD]
        qo.append(q1 * c - q2 * s)
        qo.append(q2 * c + q1 * s)
    q_ref[...] = jnp.concatenate(qo, axis=1)
    k1 = k[:, :HD // 2]
    k2 = k[:, HD // 2:]
    k_ref[...] = jnp.concatenate([k1 * c - k2 * s, k2 * c + k1 * s], axis=1)
    v_ref[...] = v


def _stage1(x, qwp, kwp, vw, g, b, cos, sin):
    n = S // BLK
    return pl.pallas_call(
        _s1_body,
        grid=(n,),
        in_specs=[
            pl.BlockSpec((BLK, H), lambda i: (i, 0)),
            pl.BlockSpec((H, H), lambda i: (0, 0)),
            pl.BlockSpec((H, HD), lambda i: (0, 0)),
            pl.BlockSpec((H, HD), lambda i: (0, 0)),
            pl.BlockSpec((1, H), lambda i: (0, 0)),
            pl.BlockSpec((1, H), lambda i: (0, 0)),
            pl.BlockSpec((BLK, HD // 2), lambda i: (i, 0)),
            pl.BlockSpec((BLK, HD // 2), lambda i: (i, 0)),
        ],
        out_specs=[
            pl.BlockSpec((BLK, H), lambda i: (i, 0)),
            pl.BlockSpec((BLK, HD), lambda i: (i, 0)),
            pl.BlockSpec((BLK, HD), lambda i: (i, 0)),
        ],
        out_shape=[
            jax.ShapeDtypeStruct((S, H), jnp.float32),
            jax.ShapeDtypeStruct((S, HD), jnp.float32),
            jax.ShapeDtypeStruct((S, HD), jnp.float32),
        ],
    )(x, qwp, kwp, vw, g, b, cos, sin)


def _attn_body(q_ref, k_ref, v_ref, o_ref):
    q = q_ref[...]
    k = k_ref[...]
    s = lax.dot_general(q, k, (((1,), (1,)), ((), ())),
                        preferred_element_type=jnp.float32) * (HD ** -0.5)
    m = jnp.max(s, axis=1, keepdims=True)
    p = jnp.exp(s - m)
    l = jnp.sum(p, axis=1, keepdims=True)
    o = jnp.dot(p, v_ref[...], preferred_element_type=jnp.float32)
    o_ref[...] = o / l


def _attn(q, k, v):
    n = S // BLK
    return pl.pallas_call(
        _attn_body,
        grid=(NH, n),
        in_specs=[
            pl.BlockSpec((BLK, HD), lambda h, i: (i, h)),
            pl.BlockSpec((S, HD), lambda h, i: (0, 0)),
            pl.BlockSpec((S, HD), lambda h, i: (0, 0)),
        ],
        out_specs=pl.BlockSpec((BLK, HD), lambda h, i: (i, h)),
        out_shape=jax.ShapeDtypeStruct((S, H), jnp.float32),
    )(q, k, v)


def _s3_body(a_ref, ow_ref, r_ref, g_ref, b_ref, rw_ref,
             h2_ref, hn_ref, w_ref):
    ao = jnp.dot(a_ref[...], ow_ref[...], preferred_element_type=jnp.float32)
    h2 = r_ref[...] + ao
    h2_ref[...] = h2
    mu = jnp.mean(h2, axis=1, keepdims=True)
    var = jnp.mean((h2 - mu) ** 2, axis=1, keepdims=True)
    hn = (h2 - mu) * lax.rsqrt(var + 1e-5) * g_ref[...] + b_ref[...]
    hn_ref[...] = hn
    logits = jnp.dot(hn, rw_ref[...], preferred_element_type=jnp.float32)
    lane = lax.broadcasted_iota(jnp.int32, logits.shape, 1)
    valid = lane < E
    logits = jnp.where(valid, logits, -1e30)
    m = jnp.max(logits, axis=1, keepdims=True)
    ex = jnp.where(valid, jnp.exp(logits - m), 0.0)
    probs = ex / jnp.sum(ex, axis=1, keepdims=True)
    m1 = jnp.max(probs, axis=1, keepdims=True)
    i1 = jnp.min(jnp.where(probs == m1, lane, 128), axis=1, keepdims=True)
    p2 = jnp.where(lane == i1, -1.0, probs)
    m2 = jnp.max(p2, axis=1, keepdims=True)
    i2 = jnp.min(jnp.where(p2 == m2, lane, 128), axis=1, keepdims=True)
    tot = m1 + m2
    w1 = m1 / tot
    w2 = m2 / tot
    w_ref[...] = jnp.where(lane == i1, w1, 0.0) + jnp.where(lane == i2, w2, 0.0)


def _stage3(attn_out, ow, resid, g, b, rwp):
    n = S // BLK
    return pl.pallas_call(
        _s3_body,
        grid=(n,),
        in_specs=[
            pl.BlockSpec((BLK, H), lambda i: (i, 0)),
            pl.BlockSpec((H, H), lambda i: (0, 0)),
            pl.BlockSpec((BLK, H), lambda i: (i, 0)),
            pl.BlockSpec((1, H), lambda i: (0, 0)),
            pl.BlockSpec((1, H), lambda i: (0, 0)),
            pl.BlockSpec((H, 128), lambda i: (0, 0)),
        ],
        out_specs=[
            pl.BlockSpec((BLK, H), lambda i: (i, 0)),
            pl.BlockSpec((BLK, H), lambda i: (i, 0)),
            pl.BlockSpec((BLK, 128), lambda i: (i, 0)),
        ],
        out_shape=[
            jax.ShapeDtypeStruct((S, H), jnp.float32),
            jax.ShapeDtypeStruct((S, H), jnp.float32),
            jax.ShapeDtypeStruct((S, 128), jnp.float32),
        ],
    )(attn_out, ow, resid, g, b, rwp)


_CH = 32
_RPW = P // _NW
_NCH = _RPW // _CH


def _sc_gather(src, chunk_act, table):
    mesh = plsc.VectorSubcoreMesh(core_axis_name="c", subcore_axis_name="s")

    @functools.partial(
        pl.kernel, mesh=mesh,
        out_type=jax.ShapeDtypeStruct((P, H), jnp.float32),
        scratch_types=[
            pltpu.VMEM((_RPW,), jnp.int32),
            pltpu.VMEM((16,), jnp.int32),
            pltpu.VMEM((_CH, H), jnp.float32),
            pltpu.VMEM((_CH, H), jnp.float32),
            pltpu.SemaphoreType.DMA,
            pltpu.SemaphoreType.DMA,
        ],
    )
    def k(src_hbm, ca_hbm, table_hbm, out_hbm, idx_v, ca_v, b0, b1, gsem, wsem):
        wid = lax.axis_index("s") * _NC + lax.axis_index("c")
        base = wid * _RPW
        pltpu.sync_copy(src_hbm.at[pl.ds(base, _RPW)], idx_v)
        pltpu.sync_copy(ca_hbm.at[pl.ds(wid * 16, 16)], ca_v)
        cav = ca_v[...]
        flags = [cav[c] for c in range(_NCH)]
        bufs = [b0, b1]

        def g_cp(c):
            return pltpu.make_async_copy(
                table_hbm.at[idx_v.at[pl.ds(c * _CH, _CH)]], bufs[c % 2], gsem)

        def w_cp(c):
            return pltpu.make_async_copy(
                bufs[c % 2], out_hbm.at[pl.ds(base + c * _CH, _CH)], wsem)

        @pl.when(flags[0] == 1)
        def _():
            g_cp(0).start()

        for c in range(_NCH):
            @pl.when(flags[c] == 1)
            def _(c=c):
                g_cp(c).wait()
            if c >= 1:
                @pl.when(flags[c - 1] == 1)
                def _(c=c):
                    w_cp(c - 1).wait()
            if c + 1 < _NCH:
                @pl.when(flags[c + 1] == 1)
                def _(c=c):
                    g_cp(c + 1).start()

            @pl.when(flags[c] == 1)
            def _(c=c):
                w_cp(c).start()

        @pl.when(flags[_NCH - 1] == 1)
        def _():
            w_cp(_NCH - 1).wait()

    return k(src, chunk_act, table)


def _sc_combine(y, resid2, pos2):
    toks_pw = S // _NW
    mesh = plsc.VectorSubcoreMesh(core_axis_name="c", subcore_axis_name="s")

    @functools.partial(
        pl.kernel, mesh=mesh,
        out_type=jax.ShapeDtypeStruct((S, H), jnp.float32),
        scratch_types=[
            pltpu.VMEM((32,), jnp.int32),
            pltpu.VMEM((32, H), jnp.float32),
            pltpu.VMEM((16, H), jnp.float32),
            pltpu.SemaphoreType.DMA,
        ],
    )
    def k(y_hbm, r_hbm, pos_hbm, out_hbm, idx_v, rows_v, acc_v, sem):
        wid = lax.axis_index("s") * _NC + lax.axis_index("c")
        for c in range(toks_pw // 16):
            t0 = wid * toks_pw + c * 16
            pltpu.sync_copy(pos_hbm.at[pl.ds(t0 * 2, 32)], idx_v)
            pltpu.async_copy(y_hbm.at[idx_v], rows_v, sem).wait()
            pltpu.sync_copy(r_hbm.at[pl.ds(t0, 16)], acc_v)

            def tok_body(i, _):
                def d_body(d, _):
                    sl = pl.ds(d * 16, 16)
                    acc_v[i, sl] = (acc_v[i, sl] + rows_v[2 * i, sl]
                                    + rows_v[2 * i + 1, sl])
                    return 0
                lax.fori_loop(0, H // 16, d_body, 0)
                return 0

            lax.fori_loop(0, 16, tok_body, 0)
            pltpu.sync_copy(acc_v, out_hbm.at[pl.ds(t0, 16)])

    return k(y, resid2, pos2)


def _moe_body(act_ref, beA_ref, jA_ref, beB_ref, jB_ref, beC_ref, jC_ref,
              x_ref, w_ref,
              ga_ref, ua_ref, da_ref,
              gb_ref, ub_ref, db_ref, cr_ref,
              gc_ref, uc_ref, dc_ref,
              y_ref, acc_ref):
    i = pl.program_id(0)
    j = pl.program_id(1)
    act = act_ref[i]

    @pl.when(act == 1)
    def _():
        x = x_ref[...]

        def fam(g_r, u_r, d_r, scale, extra):
            g = jnp.dot(x, g_r[0], preferred_element_type=jnp.float32)
            u = jnp.dot(x, u_r[0], preferred_element_type=jnp.float32)
            t = g * lax.logistic(g) * u * scale
            part = jnp.dot(t, d_r[0], preferred_element_type=jnp.float32)
            if extra is not None:
                part = part + extra

            @pl.when(j == 0)
            def _():
                acc_ref[...] = part

            @pl.when(j > 0)
            def _():
                acc_ref[...] = acc_ref[...] + part

        @pl.when(i < NB_A)
        def _():
            fam(ga_ref, ua_ref, da_ref, 1.1, None)

        @pl.when((i >= NB_A) & (i < NB_A + NB_B))
        def _():
            cre = jnp.where(
                j == 0,
                jnp.tanh(jnp.dot(x, cr_ref[0],
                                 preferred_element_type=jnp.float32)) * 0.2,
                0.0)
            fam(gb_ref, ub_ref, db_ref, 1.0, cre)

        @pl.when(i >= NB_A + NB_B)
        def _():
            fam(gc_ref, uc_ref, dc_ref, 1.0, None)

        @pl.when(j == KC - 1)
        def _():
            y_ref[...] = acc_ref[...] * w_ref[:, :1]


def _moe(xg, w2d, meta, qg_gate, qg_up, qg_down, cr_gate, cr_up, cr_down,
         cr_creative, gn_gate, gn_up, gn_down):
    act, beA, jA, beB, jB, beC, jC = meta
    grid_spec = pltpu.PrefetchScalarGridSpec(
        num_scalar_prefetch=7,
        grid=(NB, KC),
        in_specs=[
            pl.BlockSpec((BLK, H), lambda i, j, a, *p: (i * a[i], 0)),
            pl.BlockSpec((BLK, 128), lambda i, j, a, *p: (i * a[i], 0)),
            pl.BlockSpec((1, H, KA), lambda i, j, a, bA, jA, bB, jB, bC, jC: (bA[i], 0, j * jA[i])),
            pl.BlockSpec((1, H, KA), lambda i, j, a, bA, jA, bB, jB, bC, jC: (bA[i], 0, j * jA[i])),
            pl.BlockSpec((1, KA, H), lambda i, j, a, bA, jA, bB, jB, bC, jC: (bA[i], j * jA[i], 0)),
            pl.BlockSpec((1, H, KB), lambda i, j, a, bA, jA, bB, jB, bC, jC: (bB[i], 0, j * jB[i])),
            pl.BlockSpec((1, H, KB), lambda i, j, a, bA, jA, bB, jB, bC, jC: (bB[i], 0, j * jB[i])),
            pl.BlockSpec((1, KB, H), lambda i, j, a, bA, jA, bB, jB, bC, jC: (bB[i], j * jB[i], 0)),
            pl.BlockSpec((1, H, H), lambda i, j, a, bA, jA, bB, jB, bC, jC: (bB[i], 0, 0)),
            pl.BlockSpec((1, H, KB), lambda i, j, a, bA, jA, bB, jB, bC, jC: (bC[i], 0, j * jC[i])),
            pl.BlockSpec((1, H, KB), lambda i, j, a, bA, jA, bB, jB, bC, jC: (bC[i], 0, j * jC[i])),
            pl.BlockSpec((1, KB, H), lambda i, j, a, bA, jA, bB, jB, bC, jC: (bC[i], j * jC[i], 0)),
        ],
        out_specs=pl.BlockSpec((BLK, H), lambda i, j, *p: (i, 0)),
        scratch_shapes=[pltpu.VMEM((BLK, H), jnp.float32)],
    )
    return pl.pallas_call(
        _moe_body,
        grid_spec=grid_spec,
        out_shape=jax.ShapeDtypeStruct((P, H), jnp.float32),
    )(act, beA, jA, beB, jB, beC, jC,
      xg, w2d, qg_gate, qg_up, qg_down, cr_gate, cr_up, cr_down, cr_creative,
      gn_gate, gn_up, gn_down)


def _routing(w_full):
    wf = w_full[:, :E]
    sel = wf > 0.0
    seli = sel.astype(jnp.int32)
    counts = jnp.sum(seli, axis=0)
    blocks = (counts + BLK - 1) // BLK
    base = jnp.array([0, 0, NB_A, NB_A, NB_A + NB_B, 0, 0, 0], jnp.int32)
    off_blk = jnp.stack([
        base[0], blocks[0],
        base[2], base[2] + blocks[2],
        base[4], base[4] + blocks[4],
        base[4] + blocks[4] + blocks[5],
        base[4] + blocks[4] + blocks[5] + blocks[6],
    ])
    padded_off = off_blk * BLK
    rank = jnp.cumsum(seli, axis=0) - seli
    pos_te = padded_off[None, :] + rank
    tok = jnp.broadcast_to(jnp.arange(S, dtype=jnp.int32)[:, None], (S, E))
    ps = jnp.where(sel, pos_te, P).astype(jnp.int32).ravel()
    src = jnp.zeros((P + 1,), jnp.int32).at[ps].set(tok.ravel())[:P]
    w_pad = jnp.zeros((P + 1,), jnp.float32).at[ps].set(wf.ravel())[:P]
    w2d = jnp.broadcast_to(w_pad[:, None], (P, 128))
    e0 = jnp.argmax(seli, axis=1)
    e1 = (E - 1) - jnp.argmax(seli[:, ::-1], axis=1)
    pos2 = jnp.stack([
        jnp.take_along_axis(pos_te, e0[:, None], axis=1)[:, 0],
        jnp.take_along_axis(pos_te, e1[:, None], axis=1)[:, 0],
    ], axis=1).astype(jnp.int32).ravel()

    bi = jnp.arange(NB, dtype=jnp.int32)
    nA = blocks[0] + blocks[1]
    actA = (bi < nA).astype(jnp.int32)
    beA = jnp.where((bi >= blocks[0]) & (bi < nA), 1, 0).astype(jnp.int32)
    bl = bi - NB_A
    nBf = blocks[2] + blocks[3]
    actB = ((bi >= NB_A) & (bl < nBf)).astype(jnp.int32)
    beB = jnp.where(actB * (bl >= blocks[2]) == 1, 1, 0).astype(jnp.int32)
    cl = bi - NB_A - NB_B
    c4, c5, c6, c7 = blocks[4], blocks[5], blocks[6], blocks[7]
    nC = c4 + c5 + c6 + c7
    actC = ((bi >= NB_A + NB_B) & (cl < nC)).astype(jnp.int32)
    beC = ((cl >= c4).astype(jnp.int32) + (cl >= c4 + c5).astype(jnp.int32)
           + (cl >= c4 + c5 + c6).astype(jnp.int32)) * actC
    act = actA + actB + actC
    jA = actA
    jB = actB
    jC = actC

    nA_r, nB_r, nC_r = nA * BLK, nBf * BLK, nC * BLK
    offs = (jnp.arange(_NW, dtype=jnp.int32)[:, None] * _RPW
            + jnp.arange(16, dtype=jnp.int32)[None, :] * _CH)
    rA, rB, rC = NB_A * BLK, (NB_A + NB_B) * BLK, NB * BLK
    ca = jnp.where(offs < rA, offs < nA_r,
                   jnp.where(offs < rB, offs - rA < nB_r, offs - rB < nC_r))
    ca = (ca & (jnp.arange(16)[None, :] < _NCH)).astype(jnp.int32).ravel()
    return src, w2d, pos2, ca, (act, beA, jA, beB, jB, beC, jC)


def kernel(hidden_states, q_w, k_w, v_w, o_w, ln1_g, ln1_b, ln2_g, ln2_b,
           router_w, qg_gate, qg_up, qg_down, cr_gate, cr_up, cr_down,
           cr_creative, gn_gate, gn_up, gn_down):
    x = hidden_states.reshape(S, H)
    qwp = q_w[:, _rope_perm()]
    kwp = k_w[:, _rope_perm_k()]
    freqs = 1.0 / THETA ** (jnp.arange(0, HD, 2, dtype=jnp.float32) / HD)
    t = jnp.arange(S, dtype=jnp.float32)
    ang = jnp.outer(t, freqs)
    cos = jnp.cos(ang)
    sin = jnp.sin(ang)
    rwp = jnp.pad(router_w, ((0, 0), (0, 128 - E)))

    q, k, v = _stage1(x, qwp, kwp, v_w, ln1_g[None, :], ln1_b[None, :], cos, sin)
    attn_out = _attn(q, k, v)
    h2, hn2, w_full = _stage3(attn_out, o_w, x, ln2_g[None, :], ln2_b[None, :], rwp)
    src, w2d, pos2, ca, meta = _routing(w_full)
    xg = _sc_gather(src, ca, hn2)
    y = _moe(xg, w2d, meta, qg_gate, qg_up, qg_down, cr_gate, cr_up, cr_down,
             cr_creative, gn_gate, gn_up, gn_down)
    out = _sc_combine(y, h2, pos2)
    return out.reshape(1, S, H)
